# Initial kernel scaffold; baseline (speedup 1.0000x reference)
import jax
import jax.numpy as jnp
from jax.experimental import pallas as pl


def kernel(x, params, edge_index):
    raise NotImplementedError("write your pallas kernel here")


# speedup vs baseline: 3.2486x; 3.2486x over previous
import functools

import jax
import jax.numpy as jnp
from jax import lax
from jax.experimental import pallas as pl
from jax.experimental.pallas import tpu as pltpu
from jax.experimental.pallas import tpu_sc as plsc

N = 10000
E = 320000
DH = 256
HALF = DH // 2
NC = 2
NS = 16
CH = 80
EPS = E // NS
NCHUNK = EPS // CH
EPS2 = E // (NC * NS)
NCHUNK2 = EPS2 // CH
NP = 10240
RPS = NP // NS


@functools.lru_cache(maxsize=None)
def _build_sc_kernels():
    mesh = plsc.VectorSubcoreMesh(core_axis_name="c", subcore_axis_name="s")

    @functools.partial(
        pl.kernel,
        mesh=mesh,
        out_type=jax.ShapeDtypeStruct((NC * NP, 128), jnp.float32),
        scratch_types=[
            pltpu.VMEM((CH,), jnp.int32),
            pltpu.VMEM((CH, 128), jnp.float32),
            pltpu.VMEM_SHARED((NP, 128), jnp.float32),
        ],
    )
    def _sc_degree(dst_hbm, zero_hbm, ones_hbm, out_hbm, dstv, ones_v, acc):
        c = lax.axis_index("c")
        s = lax.axis_index("s")
        pltpu.sync_copy(ones_hbm, ones_v)
        pltpu.sync_copy(zero_hbm, acc.at[pl.ds(s * RPS, RPS)])
        plsc.subcore_barrier()
        base0 = (c * NS + s) * EPS2

        @pl.loop(0, NCHUNK2)
        def _(i):
            pltpu.sync_copy(dst_hbm.at[pl.ds(base0 + i * CH, CH)], dstv)
            pltpu.sync_copy(ones_v, acc.at[dstv], add=True)

        plsc.subcore_barrier()
        pltpu.sync_copy(acc.at[pl.ds(s * RPS, RPS)],
                        out_hbm.at[pl.ds(c * NP + s * RPS, RPS)])

    @functools.partial(
        pl.kernel,
        mesh=mesh,
        out_type=[jax.ShapeDtypeStruct((NP, HALF), jnp.float32),
                  jax.ShapeDtypeStruct((NP, HALF), jnp.float32)],
        scratch_types=[
            pltpu.VMEM((CH,), jnp.int32),
            pltpu.VMEM((CH,), jnp.int32),
            pltpu.VMEM((CH, HALF), jnp.float32),
            pltpu.VMEM_SHARED((NP, HALF), jnp.float32),
            pltpu.SemaphoreType.DMA,
        ],
    )
    def _sc_segment_sum(h0_hbm, h1_hbm, src_hbm, dst_hbm, zero_hbm,
                        out0_hbm, out1_hbm, srcv, dstv, rows, acc, sem):
        c = lax.axis_index("c")
        s = lax.axis_index("s")
        pltpu.sync_copy(zero_hbm, acc.at[pl.ds(s * RPS, RPS)])
        plsc.subcore_barrier()

        def run(h_hbm, out_hbm):
            @pl.loop(0, NCHUNK)
            def _(i):
                base = s * EPS + i * CH
                pltpu.sync_copy(src_hbm.at[pl.ds(base, CH)], srcv)
                pltpu.sync_copy(dst_hbm.at[pl.ds(base, CH)], dstv)
                pltpu.async_copy(h_hbm.at[srcv], rows, sem).wait()
                pltpu.sync_copy(rows, acc.at[dstv], add=True)

            plsc.subcore_barrier()
            pltpu.sync_copy(acc.at[pl.ds(s * RPS, RPS)],
                            out_hbm.at[pl.ds(s * RPS, RPS)])

        @pl.when(c == 0)
        def _():
            run(h0_hbm, out0_hbm)

        @pl.when(c == 1)
        def _():
            run(h1_hbm, out1_hbm)

    return _sc_degree, _sc_segment_sum


def _proj_body(x_ref, w_ref, b_ref, h0_ref, h1_ref):
    h = jnp.dot(x_ref[...], w_ref[...],
                preferred_element_type=jnp.float32) + b_ref[...]
    h0_ref[...] = h[:, :HALF]
    h1_ref[...] = h[:, HALF:]


_proj = pl.pallas_call(
    _proj_body,
    out_shape=[jax.ShapeDtypeStruct((N, HALF), jnp.float32),
               jax.ShapeDtypeStruct((N, HALF), jnp.float32)],
)


def _sage_bn_relu(a0, a1, h0, h1, dg, wl, bl, wr, gm, bt):
    deg = jnp.maximum(dg[:N, 0:1] + dg[NP:NP + N, 0:1], 1.0)
    h = jnp.concatenate([h0[...], h1[...]], axis=1)
    agg = jnp.concatenate([a0[:N, :] / deg, a1[:N, :] / deg], axis=1)
    t = (jnp.dot(agg, wl[...], preferred_element_type=jnp.float32) + bl[...]
         + jnp.dot(h, wr[...], preferred_element_type=jnp.float32))
    mean = jnp.mean(t, axis=0, keepdims=True)
    var = jnp.mean((t - mean) ** 2, axis=0, keepdims=True)
    t = (t - mean) * lax.rsqrt(var + 1e-5) * gm[...] + bt[...]
    return jnp.maximum(t, 0.0) + h


def _layer_body(a0, a1, h0, h1, dg, wl, bl, wr, gm, bt, o0, o1):
    t = _sage_bn_relu(a0, a1, h0, h1, dg, wl, bl, wr, gm, bt)
    o0[...] = t[:, :HALF]
    o1[...] = t[:, HALF:]


_layer = pl.pallas_call(
    _layer_body,
    out_shape=[jax.ShapeDtypeStruct((N, HALF), jnp.float32),
               jax.ShapeDtypeStruct((N, HALF), jnp.float32)],
)


def _final_body(a0, a1, h0, h1, dg, wl, bl, wr, gm, bt,
                w1, b1, w2, b2, out):
    t = _sage_bn_relu(a0, a1, h0, h1, dg, wl, bl, wr, gm, bt)
    z = jnp.maximum(
        jnp.dot(t, w1[...], preferred_element_type=jnp.float32) + b1[...], 0.0)
    out[...] = jnp.dot(z, w2[...], preferred_element_type=jnp.float32) + b2[...]


_final = pl.pallas_call(
    _final_body,
    out_shape=jax.ShapeDtypeStruct((N, 8), jnp.float32),
)


def kernel(x, params, edge_index):
    ei = edge_index.astype(jnp.int32)
    src, dst = ei[0], ei[1]
    ones128 = jnp.ones((CH, 128), jnp.float32)
    zeroh = jnp.zeros((RPS, HALF), jnp.float32)

    sc_degree, sc_segment_sum = _build_sc_kernels()
    dg = sc_degree(dst, zeroh, ones128)
    h0, h1 = _proj(x, params['proj_W'], params['proj_b'].reshape(1, DH))

    out = None
    for i in range(3):
        a0, a1 = sc_segment_sum(h0, h1, src, dst, zeroh)
        args = (a0, a1, h0, h1, dg,
                params[f'conv{i}_W_l'], params[f'conv{i}_b_l'].reshape(1, DH),
                params[f'conv{i}_W_r'],
                params[f'conv{i}_gamma'].reshape(1, DH),
                params[f'conv{i}_beta'].reshape(1, DH))
        if i < 2:
            h0, h1 = _layer(*args)
        else:
            out = _final(*args,
                         params['cls_W1'], params['cls_b1'].reshape(1, HALF),
                         params['cls_W2'], params['cls_b2'].reshape(1, 8))
    return out


# speedup vs baseline: 6.8616x; 2.1122x over previous
import functools

import jax
import jax.numpy as jnp
from jax import lax
from jax.experimental import pallas as pl
from jax.experimental.pallas import tpu as pltpu
from jax.experimental.pallas import tpu_sc as plsc

N = 10000
E = 320000
DH = 256
HALF = DH // 2
NC = 2
NS = 16
CH = 128
NCHUNK = -(-E // (NS * CH))
NCHUNKP = NCHUNK + 1
NCHUNK2 = -(-E // (NC * NS * CH))
EPAD = NS * NCHUNKP * CH
NP = 10240
RPS = NP // NS


@functools.lru_cache(maxsize=None)
def _build_sc_kernels():
    mesh = plsc.VectorSubcoreMesh(core_axis_name="c", subcore_axis_name="s")

    @functools.partial(
        pl.kernel,
        mesh=mesh,
        out_type=jax.ShapeDtypeStruct((NC * NP, 128), jnp.float32),
        scratch_types=[
            pltpu.VMEM((NCHUNK2, CH), jnp.int32),
            pltpu.VMEM((CH, 128), jnp.float32),
            pltpu.VMEM_SHARED((NP, 128), jnp.float32),
        ],
    )
    def _sc_degree(dst_hbm, zero_hbm, ones_hbm, out_hbm, dstv, ones_v, acc):
        c = lax.axis_index("c")
        s = lax.axis_index("s")
        pltpu.sync_copy(ones_hbm, ones_v)
        pltpu.sync_copy(dst_hbm.at[c * NS + s], dstv)
        pltpu.sync_copy(zero_hbm, acc.at[pl.ds(s * RPS, RPS)])
        plsc.subcore_barrier()

        @pl.loop(0, NCHUNK2)
        def _(i):
            pltpu.sync_copy(ones_v, acc.at[dstv.at[i]], add=True)

        plsc.subcore_barrier()
        pltpu.sync_copy(acc.at[pl.ds(s * RPS, RPS)],
                        out_hbm.at[pl.ds(c * NP + s * RPS, RPS)])

    @functools.partial(
        pl.kernel,
        mesh=mesh,
        out_type=[jax.ShapeDtypeStruct((NP, HALF), jnp.float32),
                  jax.ShapeDtypeStruct((NP, HALF), jnp.float32)],
        scratch_types=[
            pltpu.VMEM((CH,), jnp.int32),
            pltpu.VMEM((CH,), jnp.int32),
            pltpu.VMEM((CH,), jnp.int32),
            pltpu.VMEM((CH,), jnp.int32),
            pltpu.VMEM((CH, HALF), jnp.float32),
            pltpu.VMEM((CH, HALF), jnp.float32),
            pltpu.VMEM_SHARED((NP, HALF), jnp.float32),
            pltpu.SemaphoreType.DMA,
            pltpu.SemaphoreType.DMA,
            pltpu.SemaphoreType.DMA,
            pltpu.SemaphoreType.DMA,
            pltpu.SemaphoreType.DMA,
            pltpu.SemaphoreType.DMA,
        ],
    )
    def _sc_segment_sum(h0_hbm, h1_hbm, src_hbm, dst_hbm, zero_hbm,
                        out0_hbm, out1_hbm, srcv0, srcv1, dstv0, dstv1,
                        rows0, rows1, acc,
                        gsem0, gsem1, ssem0, ssem1, dsem0, dsem1):
        c = lax.axis_index("c")
        s = lax.axis_index("s")
        base = s * NCHUNKP
        srcv = (srcv0, srcv1)
        dstv = (dstv0, dstv1)
        rows = (rows0, rows1)
        gsem = (gsem0, gsem1)
        ssem = (ssem0, ssem1)
        dsem = (dsem0, dsem1)

        def sload(i, k):
            pltpu.async_copy(src_hbm.at[base + i], srcv[k], ssem[k])

        def dload(i, k):
            pltpu.async_copy(dst_hbm.at[base + i], dstv[k], dsem[k])

        def swaiti(k):
            pltpu.make_async_copy(src_hbm.at[base], srcv[k], ssem[k]).wait()

        def dwaiti(k):
            pltpu.make_async_copy(dst_hbm.at[base], dstv[k], dsem[k]).wait()

        sload(0, 0)
        dload(0, 0)
        sload(1, 1)
        dload(1, 1)
        pltpu.sync_copy(zero_hbm, acc.at[pl.ds(s * RPS, RPS)])
        plsc.subcore_barrier()

        def run(h_hbm, out_hbm):
            def gather(k):
                pltpu.async_copy(h_hbm.at[srcv[k]], rows[k], gsem[k])

            def gwait(k):
                pltpu.make_async_copy(h_hbm.at[srcv[k]], rows[k],
                                      gsem[k]).wait()

            def scatter(k):
                pltpu.sync_copy(rows[k], acc.at[dstv[k]], add=True)

            swaiti(0)
            gather(0)

            @pl.loop(0, (NCHUNK - 1) // 2)
            def _(j):
                i = 2 * j
                swaiti(1)
                gather(1)
                gwait(0)
                sload(i + 2, 0)
                dwaiti(0)
                scatter(0)
                dload(i + 2, 0)
                swaiti(0)
                gather(0)
                gwait(1)
                sload(i + 3, 1)
                dwaiti(1)
                scatter(1)
                dload(i + 3, 1)

            gwait(0)
            dwaiti(0)
            scatter(0)
            swaiti(1)
            dwaiti(1)

            plsc.subcore_barrier()
            pltpu.sync_copy(acc.at[pl.ds(s * RPS, RPS)],
                            out_hbm.at[pl.ds(s * RPS, RPS)])

        @pl.when(c == 0)
        def _():
            run(h0_hbm, out0_hbm)

        @pl.when(c == 1)
        def _():
            run(h1_hbm, out1_hbm)

    return _sc_degree, _sc_segment_sum


def _proj_body(x_ref, w_ref, b_ref, h0_ref, h1_ref):
    h = jnp.dot(x_ref[...], w_ref[...],
                preferred_element_type=jnp.float32) + b_ref[...]
    h0_ref[...] = h[:, :HALF]
    h1_ref[...] = h[:, HALF:]


_proj = pl.pallas_call(
    _proj_body,
    out_shape=[jax.ShapeDtypeStruct((N, HALF), jnp.float32),
               jax.ShapeDtypeStruct((N, HALF), jnp.float32)],
)


def _sage_bn_relu(a0, a1, h0, h1, dg, wl, bl, wr, gm, bt):
    deg = jnp.maximum(dg[:N, 0:1] + dg[NP:NP + N, 0:1], 1.0)
    h = jnp.concatenate([h0[...], h1[...]], axis=1)
    agg = jnp.concatenate([a0[:N, :] / deg, a1[:N, :] / deg], axis=1)
    t = (jnp.dot(agg, wl[...], preferred_element_type=jnp.float32) + bl[...]
         + jnp.dot(h, wr[...], preferred_element_type=jnp.float32))
    mean = jnp.mean(t, axis=0, keepdims=True)
    var = jnp.mean((t - mean) ** 2, axis=0, keepdims=True)
    t = (t - mean) * lax.rsqrt(var + 1e-5) * gm[...] + bt[...]
    return jnp.maximum(t, 0.0) + h


def _layer_body(a0, a1, h0, h1, dg, wl, bl, wr, gm, bt, o0, o1):
    t = _sage_bn_relu(a0, a1, h0, h1, dg, wl, bl, wr, gm, bt)
    o0[...] = t[:, :HALF]
    o1[...] = t[:, HALF:]


_layer = pl.pallas_call(
    _layer_body,
    out_shape=[jax.ShapeDtypeStruct((N, HALF), jnp.float32),
               jax.ShapeDtypeStruct((N, HALF), jnp.float32)],
)


def _final_body(a0, a1, h0, h1, dg, wl, bl, wr, gm, bt,
                w1, b1, w2, b2, out):
    t = _sage_bn_relu(a0, a1, h0, h1, dg, wl, bl, wr, gm, bt)
    z = jnp.maximum(
        jnp.dot(t, w1[...], preferred_element_type=jnp.float32) + b1[...], 0.0)
    out[...] = jnp.dot(z, w2[...], preferred_element_type=jnp.float32) + b2[...]


_final = pl.pallas_call(
    _final_body,
    out_shape=jax.ShapeDtypeStruct((N, 8), jnp.float32),
)


def kernel(x, params, edge_index):
    ei = edge_index.astype(jnp.int32)
    src, dst = ei[0], ei[1]
    nproc = NS * NCHUNK * CH
    src_pad = jnp.pad(
        jnp.concatenate([src, jnp.zeros((nproc - E,), jnp.int32)]
                        ).reshape(NS, NCHUNK * CH),
        ((0, 0), (0, CH))).reshape(NS, NCHUNKP, CH)
    dst_pad = jnp.pad(
        jnp.concatenate([dst, jnp.full((nproc - E,), NP - 1, jnp.int32)]
                        ).reshape(NS, NCHUNK * CH),
        ((0, 0), (0, CH)), constant_values=NP - 1).reshape(NS, NCHUNKP, CH)
    src_seg = src_pad.reshape(NS * NCHUNKP, CH)
    dst_seg = dst_pad.reshape(NS * NCHUNKP, CH)
    dst_deg = dst_pad.reshape(NC * NS, NCHUNK2, CH)
    ones128 = jnp.ones((CH, 128), jnp.float32)
    zeroh = jnp.zeros((RPS, HALF), jnp.float32)

    sc_degree, sc_segment_sum = _build_sc_kernels()
    dg = sc_degree(dst_deg, zeroh, ones128)
    h0, h1 = _proj(x, params['proj_W'], params['proj_b'].reshape(1, DH))

    out = None
    for i in range(3):
        a0, a1 = sc_segment_sum(h0, h1, src_seg, dst_seg, zeroh)
        args = (a0, a1, h0, h1, dg,
                params[f'conv{i}_W_l'], params[f'conv{i}_b_l'].reshape(1, DH),
                params[f'conv{i}_W_r'],
                params[f'conv{i}_gamma'].reshape(1, DH),
                params[f'conv{i}_beta'].reshape(1, DH))
        if i < 2:
            h0, h1 = _layer(*args)
        else:
            out = _final(*args,
                         params['cls_W1'], params['cls_b1'].reshape(1, HALF),
                         params['cls_W2'], params['cls_b2'].reshape(1, 8))
    return out
